# Initial kernel scaffold; baseline (speedup 1.0000x reference)
import jax
import jax.numpy as jnp
from jax.experimental import pallas as pl


def kernel(sample_1, sample_2, labels, original_target):
    raise NotImplementedError("write your pallas kernel here")


# speedup vs baseline: 3.9843x; 3.9843x over previous
import jax
import jax.numpy as jnp
from jax import lax
from jax.experimental import pallas as pl

B = 16384
D = 1024
POS_WEIGHT = 2.0
EPS = 1e-8
K = 20
ROWS_PER_BLOCK = 1024
NUM_BLOCKS = B // ROWS_PER_BLOCK
SEL_ROWS = 16
SEL_COLS = B // SEL_ROWS
BIG = 1 << 30


def _sim_kernel(a_ref, b_ref, o_ref):
    a = a_ref[...]
    b = b_ref[...]
    num = jnp.sum(a * b, axis=1, keepdims=True)
    na = jnp.sqrt(jnp.sum(a * a, axis=1, keepdims=True))
    nb = jnp.sqrt(jnp.sum(b * b, axis=1, keepdims=True))
    o_ref[...] = num / jnp.maximum(na * nb, EPS)


def _loss_kernel(sim_ref, lab_ref, o_ref):
    sim = sim_ref[...]
    lab = lab_ref[...]
    neg = jnp.where(lab == 0.0, sim, -3.0)
    pos = jnp.where(lab == 0.0, 3.0, sim)
    r = lax.broadcasted_iota(jnp.int32, (SEL_ROWS, SEL_COLS), 0)
    c = lax.broadcasted_iota(jnp.int32, (SEL_ROWS, SEL_COLS), 1)
    flat = r * SEL_COLS + c

    def body_neg(_, carry):
        v, tot = carry
        m = jnp.max(v)
        sel = jnp.min(jnp.where(v == m, flat, BIG))
        v = jnp.where(flat == sel, -3.0, v)
        return v, tot + m * m

    _, tot_n = lax.fori_loop(0, K, body_neg, (neg, jnp.float32(0.0)))

    def body_pos(_, carry):
        v, tot = carry
        m = jnp.min(v)
        sel = jnp.min(jnp.where(v == m, flat, BIG))
        v = jnp.where(flat == sel, 3.0, v)
        d = m - 1.0
        return v, tot + d * d

    _, tot_p = lax.fori_loop(0, K, body_pos, (pos, jnp.float32(0.0)))
    o_ref[...] = jnp.broadcast_to((tot_n + tot_p) * (1.0 / (2 * K)), (1, 1))


def kernel(sample_1, sample_2, labels, original_target):
    sim = pl.pallas_call(
        _sim_kernel,
        grid=(NUM_BLOCKS,),
        in_specs=[
            pl.BlockSpec((ROWS_PER_BLOCK, D), lambda i: (i, 0)),
            pl.BlockSpec((ROWS_PER_BLOCK, D), lambda i: (i, 0)),
        ],
        out_specs=pl.BlockSpec((ROWS_PER_BLOCK, 1), lambda i: (i, 0)),
        out_shape=jax.ShapeDtypeStruct((B, 1), jnp.float32),
    )(sample_1, sample_2)

    sim2d = sim.reshape(SEL_ROWS, SEL_COLS)
    lab2d = labels.reshape(SEL_ROWS, SEL_COLS)
    loss = pl.pallas_call(
        _loss_kernel,
        out_shape=jax.ShapeDtypeStruct((1, 1), jnp.float32),
    )(sim2d, lab2d)

    weight = (POS_WEIGHT - 1.0) * jnp.float32(original_target) + 1.0
    return loss[0, 0] * weight


# speedup vs baseline: 4.4424x; 1.1150x over previous
import jax
import jax.numpy as jnp
from jax import lax
from jax.experimental import pallas as pl

B = 16384
D = 1024
POS_WEIGHT = 2.0
EPS = 1e-8
K = 20
ROWS_PER_BLOCK = 1024
NUM_BLOCKS = B // ROWS_PER_BLOCK
SEL_ROWS = 128
SEL_COLS = B // SEL_ROWS
BIG = 1 << 30


def _sim_kernel(a_ref, b_ref, o_ref):
    a = a_ref[...]
    b = b_ref[...]
    num = jnp.sum(a * b, axis=1, keepdims=True)
    na = jnp.sqrt(jnp.sum(a * a, axis=1, keepdims=True))
    nb = jnp.sqrt(jnp.sum(b * b, axis=1, keepdims=True))
    o_ref[...] = num / jnp.maximum(na * nb, EPS)


def _loss_kernel(sim_ref, lab_ref, o_ref):
    sim = sim_ref[...]
    lab = lab_ref[...]
    neg = jnp.where(lab == 0.0, sim, -3.0)
    pos = jnp.where(lab == 0.0, 3.0, sim)
    r = lax.broadcasted_iota(jnp.int32, (SEL_ROWS, SEL_COLS), 0)
    c = lax.broadcasted_iota(jnp.int32, (SEL_ROWS, SEL_COLS), 1)
    flat = r * SEL_COLS + c

    def body(_, carry):
        vn, vp, tot_n, tot_p = carry
        mn = jnp.max(vn)
        mp = jnp.min(vp)
        sel_n = jnp.min(jnp.where(vn == mn, flat, BIG))
        sel_p = jnp.min(jnp.where(vp == mp, flat, BIG))
        vn = jnp.where(flat == sel_n, -3.0, vn)
        vp = jnp.where(flat == sel_p, 3.0, vp)
        d = mp - 1.0
        return vn, vp, tot_n + mn * mn, tot_p + d * d

    _, _, tot_n, tot_p = lax.fori_loop(
        0, K, body, (neg, pos, jnp.float32(0.0), jnp.float32(0.0))
    )
    o_ref[...] = jnp.broadcast_to((tot_n + tot_p) * (1.0 / (2 * K)), (1, 1))


def kernel(sample_1, sample_2, labels, original_target):
    sim = pl.pallas_call(
        _sim_kernel,
        grid=(NUM_BLOCKS,),
        in_specs=[
            pl.BlockSpec((ROWS_PER_BLOCK, D), lambda i: (i, 0)),
            pl.BlockSpec((ROWS_PER_BLOCK, D), lambda i: (i, 0)),
        ],
        out_specs=pl.BlockSpec((ROWS_PER_BLOCK, 1), lambda i: (i, 0)),
        out_shape=jax.ShapeDtypeStruct((B, 1), jnp.float32),
    )(sample_1, sample_2)

    sim2d = sim.reshape(SEL_ROWS, SEL_COLS)
    lab2d = labels.reshape(SEL_ROWS, SEL_COLS)
    loss = pl.pallas_call(
        _loss_kernel,
        out_shape=jax.ShapeDtypeStruct((1, 1), jnp.float32),
    )(sim2d, lab2d)

    weight = (POS_WEIGHT - 1.0) * jnp.float32(original_target) + 1.0
    return loss[0, 0] * weight
